# Initial kernel scaffold; baseline (speedup 1.0000x reference)
import jax
import jax.numpy as jnp
from jax.experimental import pallas as pl


def kernel(node_feats, edge_index, W_self, W_neigh, bias, W_res, b_res, gamma, beta):
    raise NotImplementedError("write your pallas kernel here")


# speedup vs baseline: 5.5591x; 5.5591x over previous
import jax
import jax.numpy as jnp
from jax import lax
from jax.experimental import pallas as pl
from jax.experimental.pallas import tpu as pltpu
from jax.experimental.pallas import tpu_sc as plsc

_N = 10000
_E = 320000
_D = 128

_CHUNK = 128
_NCHUNKS = _E // _CHUNK
_NC = 2
_NS = 16
_NW = _NC * _NS
_RPT = 632
_NPAD = _RPT * _NS
_TAIL = _RPT - 4 * _CHUNK
_DEGW = 128

_mesh = lambda: plsc.VectorSubcoreMesh(core_axis_name="c", subcore_axis_name="s",
                                       num_cores=_NC, num_subcores=_NS)


def _init_acc(zsrc, acc_sh, base):
    for t in range(4):
        pltpu.sync_copy(zsrc, acc_sh.at[pl.ds(base + t * _CHUNK, _CHUNK)])
    pltpu.sync_copy(zsrc.at[pl.ds(0, _TAIL)],
                    acc_sh.at[pl.ds(base + 4 * _CHUNK, _TAIL)])


def _copy_out(acc_sh, out, c, base):
    for t in range(4):
        off = base + t * _CHUNK
        pltpu.sync_copy(acc_sh.at[pl.ds(off, _CHUNK)], out.at[c, pl.ds(off, _CHUNK)])
    tail = base + 4 * _CHUNK
    pltpu.sync_copy(acc_sh.at[pl.ds(tail, _TAIL)], out.at[c, pl.ds(tail, _TAIL)])


def _nloc(wid):
    return _NCHUNKS // _NW + jnp.where(wid < _NCHUNKS % _NW, 1, 0)


def _feat_body(src_hbm, dst_hbm, node_hbm, zeros_hbm, parts_out,
               src_idx, dst_idx, rows, acc_sh, sem):
    c = lax.axis_index("c")
    s = lax.axis_index("s")
    wid = s * _NC + c

    base = s * _RPT
    _init_acc(zeros_hbm, acc_sh, base)
    plsc.subcore_barrier()

    def step(t, _):
        e0 = (wid + t * _NW) * _CHUNK
        pltpu.sync_copy(src_hbm.at[pl.ds(e0, _CHUNK)], src_idx)
        pltpu.sync_copy(dst_hbm.at[pl.ds(e0, _CHUNK)], dst_idx)
        pltpu.async_copy(node_hbm.at[src_idx], rows, sem).wait()
        pltpu.sync_copy(rows, acc_sh.at[dst_idx], add=True)
        return 0
    lax.fori_loop(0, _nloc(wid), step, 0)
    plsc.subcore_barrier()

    _copy_out(acc_sh, parts_out, c, base)


def _deg_body(dst_hbm, ones_hbm, zeros_hbm, degp_out, dst_idx, ones, deg_sh):
    c = lax.axis_index("c")
    s = lax.axis_index("s")
    wid = s * _NC + c

    pltpu.sync_copy(ones_hbm, ones)
    base = s * _RPT
    _init_acc(zeros_hbm, deg_sh, base)
    plsc.subcore_barrier()

    def step(t, _):
        e0 = (wid + t * _NW) * _CHUNK
        pltpu.sync_copy(dst_hbm.at[pl.ds(e0, _CHUNK)], dst_idx)
        pltpu.sync_copy(ones, deg_sh.at[dst_idx], add=True)
        return 0
    lax.fori_loop(0, _nloc(wid), step, 0)
    plsc.subcore_barrier()

    _copy_out(deg_sh, degp_out, c, base)


def _sc_aggregate(src, dst, node_feats):
    feat = pl.kernel(
        _feat_body,
        out_type=jax.ShapeDtypeStruct((_NC, _NPAD, _D), jnp.float32),
        mesh=_mesh(),
        scratch_types=[
            pltpu.VMEM((_CHUNK,), jnp.int32),
            pltpu.VMEM((_CHUNK,), jnp.int32),
            pltpu.VMEM((_CHUNK, _D), jnp.float32),
            pltpu.VMEM_SHARED((_NPAD, _D), jnp.float32),
            pltpu.SemaphoreType.DMA,
        ],
    )
    deg = pl.kernel(
        _deg_body,
        out_type=jax.ShapeDtypeStruct((_NC, _NPAD, _DEGW), jnp.float32),
        mesh=_mesh(),
        scratch_types=[
            pltpu.VMEM((_CHUNK,), jnp.int32),
            pltpu.VMEM((_CHUNK, _DEGW), jnp.float32),
            pltpu.VMEM_SHARED((_NPAD, _DEGW), jnp.float32),
        ],
    )
    zeros128 = jnp.zeros((_CHUNK, _D), jnp.float32)
    ones128 = jnp.ones((_CHUNK, _DEGW), jnp.float32)
    return feat(src, dst, node_feats, zeros128), deg(dst, ones128, zeros128)


def _dense_body(x_ref, p_ref, d_ref, ws_ref, wn_ref, wr_ref,
                bias_ref, bres_ref, gamma_ref, beta_ref, out_ref):
    x = x_ref[...]
    agg = p_ref[0, :_N] + p_ref[1, :_N]
    deg = d_ref[0, :_N, 0:1] + d_ref[1, :_N, 0:1]
    h_neigh = agg / jnp.maximum(deg, 1.0)
    rst = (jnp.dot(x, ws_ref[...], preferred_element_type=jnp.float32)
           + jnp.dot(h_neigh, wn_ref[...], preferred_element_type=jnp.float32)
           + bias_ref[...])
    res = jnp.maximum(
        jnp.dot(x, wr_ref[...], preferred_element_type=jnp.float32) + bres_ref[...],
        0.0)
    h = rst + res
    mean = jnp.mean(h, axis=0, keepdims=True)
    var = jnp.mean((h - mean) ** 2, axis=0, keepdims=True)
    out_ref[...] = ((h - mean) * lax.rsqrt(var + 1e-5)) * gamma_ref[...] + beta_ref[...]


@jax.jit
def kernel(node_feats, edge_index, W_self, W_neigh, bias, W_res, b_res, gamma, beta):
    parts, degp = _sc_aggregate(edge_index[0], edge_index[1], node_feats)
    return pl.pallas_call(
        _dense_body,
        out_shape=jax.ShapeDtypeStruct((_N, _D), jnp.float32),
    )(node_feats, parts, degp, W_self, W_neigh, W_res,
      bias.reshape(1, _D), b_res.reshape(1, _D),
      gamma.reshape(1, _D), beta.reshape(1, _D))
